# Initial kernel scaffold; baseline (speedup 1.0000x reference)
import jax
import jax.numpy as jnp
from jax.experimental import pallas as pl


def kernel(x, expert_bias, gate_w):
    raise NotImplementedError("write your pallas kernel here")


# speedup vs baseline: 1.5794x; 1.5794x over previous
import functools

import jax
import jax.numpy as jnp
from jax.experimental import pallas as pl

TOKENS = 32768
DIM = 4096
NUM_EXPERTS = 64
TOP_K = 8
BLOCK = 1024
NUM_BLOCKS = TOKENS // BLOCK


def _router_block_kernel(x_ref, gwt_ref, bias_ref, ts_ref, idx_ref, hist_ref):
    x_blk = x_ref[...]
    gwt = gwt_ref[...]
    scores = jnp.dot(x_blk, gwt, preferred_element_type=jnp.float32)
    scores = jax.nn.sigmoid(scores)
    biased = scores + bias_ref[...]

    col = jax.lax.broadcasted_iota(jnp.int32, (BLOCK, NUM_EXPERTS), 1)
    work = biased
    sel_mask = jnp.zeros((BLOCK, NUM_EXPERTS), dtype=jnp.float32)
    vals = []
    idxs = []
    for _ in range(TOP_K):
        m = jnp.max(work, axis=1, keepdims=True)
        ix = jnp.min(jnp.where(work == m, col, NUM_EXPERTS), axis=1,
                     keepdims=True)
        onehot = col == ix
        sc = jnp.sum(jnp.where(onehot, scores, 0.0), axis=1, keepdims=True)
        vals.append(sc)
        idxs.append(ix)
        sel_mask = sel_mask + onehot.astype(jnp.float32)
        work = jnp.where(onehot, -jnp.inf, work)

    top = jnp.concatenate(vals, axis=1)
    top = top / (jnp.sum(top, axis=1, keepdims=True) + 1e-20)
    ts_ref[...] = top
    idx_ref[...] = jnp.concatenate(idxs, axis=1)
    hist_ref[...] = jnp.sum(sel_mask, axis=0, keepdims=True)[None]


def _hist_reduce_kernel(parts_ref, out_ref):
    out_ref[...] = jnp.sum(parts_ref[...], axis=0)


@jax.jit
def kernel(x, expert_bias, gate_w):
    gwt = gate_w.T
    bias2d = expert_bias.reshape(1, NUM_EXPERTS)

    top_scores, indices, hist_parts = pl.pallas_call(
        _router_block_kernel,
        grid=(NUM_BLOCKS,),
        in_specs=[
            pl.BlockSpec((BLOCK, DIM), lambda i: (i, 0)),
            pl.BlockSpec((DIM, NUM_EXPERTS), lambda i: (0, 0)),
            pl.BlockSpec((1, NUM_EXPERTS), lambda i: (0, 0)),
        ],
        out_specs=[
            pl.BlockSpec((BLOCK, TOP_K), lambda i: (i, 0)),
            pl.BlockSpec((BLOCK, TOP_K), lambda i: (i, 0)),
            pl.BlockSpec((1, 1, NUM_EXPERTS), lambda i: (i, 0, 0)),
        ],
        out_shape=[
            jax.ShapeDtypeStruct((TOKENS, TOP_K), jnp.float32),
            jax.ShapeDtypeStruct((TOKENS, TOP_K), jnp.int32),
            jax.ShapeDtypeStruct((NUM_BLOCKS, 1, NUM_EXPERTS), jnp.float32),
        ],
    )(x, gwt, bias2d)

    hist = pl.pallas_call(
        _hist_reduce_kernel,
        out_shape=jax.ShapeDtypeStruct((1, NUM_EXPERTS), jnp.float32),
    )(hist_parts)

    return top_scores, indices, hist.reshape(NUM_EXPERTS)
